# Initial kernel scaffold; baseline (speedup 1.0000x reference)
import jax
import jax.numpy as jnp
from jax.experimental import pallas as pl


def kernel(edge_index, features, preference, W1, b1, W2, b2):
    raise NotImplementedError("write your pallas kernel here")


# speedup vs baseline: 8.5078x; 8.5078x over previous
import functools

import jax
import jax.numpy as jnp
from jax import lax
from jax.experimental import pallas as pl
from jax.experimental.pallas import tpu as pltpu
from jax.experimental.pallas import tpu_sc as plsc

NUM_USER = 10000
NUM_ITEM = 40000
N = NUM_USER + NUM_ITEM
DF = 128
DH = 256
D = 64
E = 800000

NC, NS = 2, 16
HALF = N // 2
DUMMY_MASK = 511
ACC_ROWS = 25600
CH = 128

_mesh = plsc.VectorSubcoreMesh(core_axis_name="c", subcore_axis_name="s",
                               num_cores=NC, num_subcores=NS)

_F32 = jnp.float32
_ZV16 = functools.partial(jnp.zeros, (16,), _F32)

EC1 = 25088
NCH1, NCH1_LAST = 196, 174


@functools.partial(
    pl.kernel,
    out_type=(jax.ShapeDtypeStruct((2 * E,), jnp.int32),
              jax.ShapeDtypeStruct((2 * N,), _F32)),
    mesh=_mesh,
    compiler_params=pltpu.CompilerParams(use_tc_tiling_on_sc=False),
    scratch_types=(pltpu.VMEM((CH,), jnp.int32),
                   pltpu.VMEM((CH,), jnp.int32),
                   pltpu.VMEM((CH,), jnp.int32),
                   pltpu.VMEM((CH,), jnp.int32),
                   pltpu.VMEM((CH,), _F32),
                   pltpu.VMEM((5008,), _F32),
                   pltpu.VMEM_SHARED((N,), _F32)),
)
def _edge_prep(edge_hbm, sidx_hbm, deg2_hbm,
               r_v, c_v, sa_v, sb_v, val_v, z_v, deg_acc):
    c = lax.axis_index("c")
    s = lax.axis_index("s")
    wid = c * NS + s

    @pl.when(s < 10)
    def _():
        def zb(g, carry):
            z_v[pl.ds(g * 16, 16)] = _ZV16()
            return carry
        lax.fori_loop(0, 313, zb, 0)
        pltpu.sync_copy(z_v.at[pl.ds(0, 5000)], deg_acc.at[pl.ds(s * 5000, 5000)])

    plsc.subcore_barrier()

    nch = jnp.where(wid == NC * NS - 1, NCH1_LAST, NCH1)
    base_e = wid * EC1

    def body(j, carry):
        e0 = base_e + j * CH
        pltpu.sync_copy(edge_hbm.at[pl.ds(e0, CH)], r_v)
        pltpu.sync_copy(edge_hbm.at[pl.ds(E + e0, CH)], c_v)
        for g in range(CH // 16):
            sl = pl.ds(g * 16, 16)
            r = r_v[sl]
            cc = c_v[sl]
            keep = r != cc
            dummy = HALF + (cc & DUMMY_MASK)
            sa_v[sl] = jnp.where(keep & (cc < HALF), cc, dummy)
            sb_v[sl] = jnp.where(keep & (cc >= HALF), cc - HALF, dummy)
            val_v[sl] = jnp.where(keep, 1.0, 0.0).astype(_F32)
        pltpu.sync_copy(sa_v, sidx_hbm.at[pl.ds(e0, CH)])
        pltpu.sync_copy(sb_v, sidx_hbm.at[pl.ds(E + e0, CH)])
        pltpu.sync_copy(val_v, deg_acc.at[r_v], add=True)
        return carry

    lax.fori_loop(0, nch, body, 0)
    plsc.subcore_barrier()

    @pl.when(s < 10)
    def _():
        sl = pl.ds(0, 5000)
        pltpu.sync_copy(deg_acc.at[pl.ds(s * 5000, 5000)], z_v.at[sl])
        pltpu.sync_copy(z_v.at[sl], deg2_hbm.at[pl.ds(c * N + s * 5000, 5000)])


RB = 400
GRID = N // RB
UB = NUM_USER // RB


def _mlp_body(pref, feat, w1, b1, w2, b2, deg_a, deg_b,
              x_out, y_out, di_out, di2_out):
    i = pl.program_id(0)

    @pl.when(i < UB)
    def _():
        x_out[...] = pref[...]

    @pl.when(i >= UB)
    def _():
        z = jnp.dot(feat[...], w1[...], preferred_element_type=_F32) + b1[...]
        z = jnp.where(z >= 0, z, 0.01 * z)
        x_out[...] = jnp.dot(z, w2[...], preferred_element_type=_F32) + b2[...]

    xb = x_out[...]
    nrm = jnp.sqrt(jnp.sum(xb * xb, axis=1, keepdims=True))
    xn = xb / jnp.maximum(nrm, 1e-12)
    x_out[...] = xn
    deg_sum = deg_a[0, 0, :] + deg_b[0, 0, :]
    dinv = jnp.where(deg_sum > 0, lax.rsqrt(deg_sum), 0.0)
    di_out[0, 0, :] = dinv
    di2_out[0, 0, :] = dinv * dinv
    y_out[...] = xn * dinv[:, None]


def _mlp(features, preference, W1, b1, W2, b2, deg2):
    deg3 = deg2.reshape(2, GRID, 1, RB)
    x, y, di, di2 = pl.pallas_call(
        _mlp_body,
        grid=(GRID,),
        in_specs=[
            pl.BlockSpec((RB, D), lambda i: (jnp.minimum(i, UB - 1), 0)),
            pl.BlockSpec((RB, DF), lambda i: (jnp.maximum(i - UB, 0), 0)),
            pl.BlockSpec((DF, DH), lambda i: (0, 0)),
            pl.BlockSpec((1, DH), lambda i: (0, 0)),
            pl.BlockSpec((DH, D), lambda i: (0, 0)),
            pl.BlockSpec((1, D), lambda i: (0, 0)),
            pl.BlockSpec((1, 1, RB), lambda i: (i, 0, 0)),
            pl.BlockSpec((1, 1, RB), lambda i: (i, 0, 0)),
        ],
        out_specs=[
            pl.BlockSpec((RB, D), lambda i: (i, 0)),
            pl.BlockSpec((RB, D), lambda i: (i, 0)),
            pl.BlockSpec((1, 1, RB), lambda i: (i, 0, 0)),
            pl.BlockSpec((1, 1, RB), lambda i: (i, 0, 0)),
        ],
        out_shape=[
            jax.ShapeDtypeStruct((N, D), _F32),
            jax.ShapeDtypeStruct((N, D), _F32),
            jax.ShapeDtypeStruct((GRID, 1, RB), _F32),
            jax.ShapeDtypeStruct((GRID, 1, RB), _F32),
        ],
    )(preference, features, W1, b1.reshape(1, DH), W2, b2.reshape(1, D),
      deg3[0], deg3[1])
    return x, y, di.reshape(N), di2.reshape(N)


EC3 = 50176
NCH3, NCH3_LAST = 392, 370
RT = 1568
RCH = RT // 16


def _zero_acc(s, zb, acc):
    def zbody(g, carry):
        zb[g // 4, pl.ds((g % 4) * 16, 16)] = _ZV16()
        return carry
    lax.fori_loop(0, 160 * 4, zbody, 0)
    for k in range(10):
        pltpu.sync_copy(zb, acc.at[pl.ds(s * 1600 + k * 160, 160)])


def _scatter_edges(src_hbm, row_hbm, sidx_hbm, cc, s, gi_v, si_v, rows_v, acc):
    nch = jnp.where(s == NS - 1, NCH3_LAST, NCH3)
    base_e = s * EC3
    half_off = cc * E

    def body(j, carry):
        e0 = base_e + j * CH
        pltpu.sync_copy(row_hbm.at[pl.ds(e0, CH)], gi_v)
        pltpu.sync_copy(sidx_hbm.at[pl.ds(half_off + e0, CH)], si_v)
        pltpu.sync_copy(src_hbm.at[gi_v], rows_v)
        pltpu.sync_copy(rows_v, acc.at[si_v], add=True)
        return carry

    lax.fori_loop(0, nch, body, 0)


@functools.partial(
    pl.kernel,
    out_type=(jax.ShapeDtypeStruct((N, D), _F32),
              jax.ShapeDtypeStruct((N, D), _F32)),
    mesh=_mesh,
    compiler_params=pltpu.CompilerParams(use_tc_tiling_on_sc=False),
    scratch_types=(pltpu.VMEM((CH,), jnp.int32),
                   pltpu.VMEM((CH,), jnp.int32),
                   pltpu.VMEM((CH, D), _F32),
                   pltpu.VMEM((160, D), _F32),
                   pltpu.VMEM((RT,), _F32),
                   pltpu.VMEM((16, D), _F32),
                   pltpu.VMEM((16, D), _F32),
                   pltpu.VMEM_SHARED((ACC_ROWS, D), _F32)),
)
def _prop1(y_hbm, row_hbm, sidx_hbm, di2_hbm, s1_hbm, y2_hbm,
           gi_v, si_v, rows_v, zb, dv, sv, yv, acc):
    cc = lax.axis_index("c")
    s = lax.axis_index("s")
    _zero_acc(s, zb, acc)
    plsc.subcore_barrier()
    _scatter_edges(y_hbm, row_hbm, sidx_hbm, cc, s, gi_v, si_v, rows_v, acc)
    plsc.subcore_barrier()

    start = jnp.minimum(s * RT, HALF - RT)
    nbase = cc * HALF + start
    pltpu.sync_copy(di2_hbm.at[pl.ds(nbase, RT)], dv)

    def body(k, carry):
        r0 = start + k * 16
        g0 = nbase + k * 16
        pltpu.sync_copy(acc.at[pl.ds(r0, 16)], sv)
        d16 = dv[pl.ds(k * 16, 16)]
        for i in range(16):
            d = d16[i]
            for q in range(D // 16):
                sl = pl.ds(q * 16, 16)
                yv[i, sl] = sv[i, sl] * d
        pltpu.sync_copy(sv, s1_hbm.at[pl.ds(g0, 16)])
        pltpu.sync_copy(yv, y2_hbm.at[pl.ds(g0, 16)])
        return carry

    lax.fori_loop(0, RCH, body, 0)


@functools.partial(
    pl.kernel,
    out_type=jax.ShapeDtypeStruct((N, D), _F32),
    mesh=_mesh,
    compiler_params=pltpu.CompilerParams(use_tc_tiling_on_sc=False),
    scratch_types=(pltpu.VMEM((CH,), jnp.int32),
                   pltpu.VMEM((CH,), jnp.int32),
                   pltpu.VMEM((CH, D), _F32),
                   pltpu.VMEM((160, D), _F32),
                   pltpu.VMEM((RT,), _F32),
                   pltpu.VMEM((16, D), _F32),
                   pltpu.VMEM((16, D), _F32),
                   pltpu.VMEM((16, D), _F32),
                   pltpu.VMEM((16, D), _F32),
                   pltpu.VMEM_SHARED((ACC_ROWS, D), _F32)),
)
def _prop2(y2_hbm, row_hbm, sidx_hbm, di_hbm, x_hbm, s1_hbm, xhat_hbm,
           gi_v, si_v, rows_v, zb, dv, sv, xv, s1v, ov, acc):
    cc = lax.axis_index("c")
    s = lax.axis_index("s")
    _zero_acc(s, zb, acc)
    plsc.subcore_barrier()
    _scatter_edges(y2_hbm, row_hbm, sidx_hbm, cc, s, gi_v, si_v, rows_v, acc)
    plsc.subcore_barrier()

    start = jnp.minimum(s * RT, HALF - RT)
    nbase = cc * HALF + start
    pltpu.sync_copy(di_hbm.at[pl.ds(nbase, RT)], dv)

    def body(k, carry):
        r0 = start + k * 16
        g0 = nbase + k * 16
        pltpu.sync_copy(acc.at[pl.ds(r0, 16)], sv)
        pltpu.sync_copy(x_hbm.at[pl.ds(g0, 16)], xv)
        pltpu.sync_copy(s1_hbm.at[pl.ds(g0, 16)], s1v)
        d16 = dv[pl.ds(k * 16, 16)]
        for i in range(16):
            d = d16[i]
            for q in range(D // 16):
                sl = pl.ds(q * 16, 16)
                ov[i, sl] = xv[i, sl] + (sv[i, sl] + s1v[i, sl]) * d
        pltpu.sync_copy(ov, xhat_hbm.at[pl.ds(g0, 16)])
        return carry

    lax.fori_loop(0, RCH, body, 0)


def kernel(edge_index, features, preference, W1, b1, W2, b2):
    edge_flat = edge_index.reshape(2 * E)
    row = edge_index[0]
    sidx, deg2 = _edge_prep(edge_flat)
    x, y, di, di2 = _mlp(features, preference, W1, b1, W2, b2, deg2)
    s1, y2 = _prop1(y, row, sidx, di2)
    x_hat = _prop2(y2, row, sidx, di, x, s1)
    return (x_hat, preference)


# speedup vs baseline: 12.6668x; 1.4888x over previous
import functools

import jax
import jax.numpy as jnp
from jax import lax
from jax.experimental import pallas as pl
from jax.experimental.pallas import tpu as pltpu
from jax.experimental.pallas import tpu_sc as plsc

NUM_USER = 10000
NUM_ITEM = 40000
N = NUM_USER + NUM_ITEM
DF = 128
DH = 256
D = 64
E = 800000

NC, NS = 2, 16
HALF = N // 2
DUMMY_MASK = 511
ACC_ROWS = 25600
CH = 128

_mesh = plsc.VectorSubcoreMesh(core_axis_name="c", subcore_axis_name="s",
                               num_cores=NC, num_subcores=NS)

_F32 = jnp.float32
_ZV16 = functools.partial(jnp.zeros, (16,), _F32)

EC1 = 25088
NCH1, NCH1_LAST = 196, 174


LCAP = EC1 + 16
LREG = 32 * EC1


@functools.partial(
    pl.kernel,
    out_type=(jax.ShapeDtypeStruct((2 * LREG,), jnp.int32),
              jax.ShapeDtypeStruct((2 * LREG,), jnp.int32),
              jax.ShapeDtypeStruct((512,), jnp.int32),
              jax.ShapeDtypeStruct((2 * N,), _F32)),
    mesh=_mesh,
    compiler_params=pltpu.CompilerParams(use_tc_tiling_on_sc=False, needs_layout_passes=False),
    scratch_types=(pltpu.VMEM((CH,), jnp.int32),
                   pltpu.VMEM((CH,), jnp.int32),
                   pltpu.VMEM((CH,), _F32),
                   pltpu.VMEM((LCAP,), jnp.int32),
                   pltpu.VMEM((LCAP,), jnp.int32),
                   pltpu.VMEM((LCAP,), jnp.int32),
                   pltpu.VMEM((LCAP,), jnp.int32),
                   pltpu.VMEM((16,), jnp.int32),
                   pltpu.VMEM((5008,), _F32),
                   pltpu.VMEM_SHARED((N,), _F32)),
)
def _edge_prep(edge_hbm, gl_hbm, sl_hbm, cnt_hbm, deg2_hbm,
               r_v, c_v, val_v, ga_v, sa_v, gb_v, sb_v, cw_v, z_v, deg_acc):
    c = lax.axis_index("c")
    s = lax.axis_index("s")
    wid = c * NS + s

    @pl.when(s < 10)
    def _():
        def zb(g, carry):
            z_v[pl.ds(g * 16, 16)] = _ZV16()
            return carry
        lax.fori_loop(0, 313, zb, 0)
        pltpu.sync_copy(z_v.at[pl.ds(0, 5000)], deg_acc.at[pl.ds(s * 5000, 5000)])

    plsc.subcore_barrier()

    nch = jnp.where(wid == NC * NS - 1, NCH1_LAST, NCH1)
    base_e = wid * EC1

    def body(j, offs):
        off_a, off_b = offs
        e0 = base_e + j * CH
        pltpu.sync_copy(edge_hbm.at[pl.ds(e0, CH)], r_v)
        pltpu.sync_copy(edge_hbm.at[pl.ds(E + e0, CH)], c_v)
        for g in range(CH // 16):
            sl = pl.ds(g * 16, 16)
            r = r_v[sl]
            cc = c_v[sl]
            keep = r != cc
            in_a = keep & (cc < HALF)
            in_b = keep & (cc >= HALF)
            cum_a = plsc.cumsum(jnp.where(in_a, 1, 0))
            cum_b = plsc.cumsum(jnp.where(in_b, 1, 0))
            dst_a = jnp.where(in_a, off_a + cum_a - 1, 0)
            dst_b = jnp.where(in_b, off_b + cum_b - 1, 0)
            plsc.store_scatter(ga_v, [dst_a], r, mask=in_a)
            plsc.store_scatter(sa_v, [dst_a], cc, mask=in_a)
            plsc.store_scatter(gb_v, [dst_b], r, mask=in_b)
            plsc.store_scatter(sb_v, [dst_b], cc - HALF, mask=in_b)
            off_a = off_a + cum_a[15]
            off_b = off_b + cum_b[15]
            val_v[sl] = jnp.where(keep, 1.0, 0.0).astype(_F32)
        pltpu.sync_copy(val_v, deg_acc.at[r_v], add=True)
        return (off_a, off_b)

    off_a, off_b = lax.fori_loop(0, nch, body, (jnp.int32(0), jnp.int32(0)))

    iota = lax.broadcasted_iota(jnp.int32, (16,), 0)

    def _pad(off, g_ref, s_ref):
        pad_to = ((off + CH - 1) // CH) * CH

        def pbody(k, o):
            g_ref[pl.ds(o, 16)] = jnp.zeros((16,), jnp.int32)
            s_ref[pl.ds(o, 16)] = HALF + ((iota * 31 + o) & DUMMY_MASK)
            return o + 16

        lax.fori_loop(0, (pad_to - off + 15) // 16, pbody, off)
        return pad_to

    cnt_a = _pad(off_a, ga_v, sa_v)
    cnt_b = _pad(off_b, gb_v, sb_v)

    base_a = wid * EC1
    base_b = LREG + wid * EC1
    pltpu.sync_copy(ga_v.at[pl.ds(0, EC1)], gl_hbm.at[pl.ds(base_a, EC1)])
    pltpu.sync_copy(sa_v.at[pl.ds(0, EC1)], sl_hbm.at[pl.ds(base_a, EC1)])
    pltpu.sync_copy(gb_v.at[pl.ds(0, EC1)], gl_hbm.at[pl.ds(base_b, EC1)])
    pltpu.sync_copy(sb_v.at[pl.ds(0, EC1)], sl_hbm.at[pl.ds(base_b, EC1)])
    cw_v[pl.ds(0, 16)] = jnp.where(iota == 0, cnt_a, jnp.where(iota == 1, cnt_b, 0))
    pltpu.sync_copy(cw_v, cnt_hbm.at[pl.ds(16 * wid, 16)])

    plsc.subcore_barrier()

    @pl.when(s < 10)
    def _():
        sl = pl.ds(0, 5000)
        pltpu.sync_copy(deg_acc.at[pl.ds(s * 5000, 5000)], z_v.at[sl])
        pltpu.sync_copy(z_v.at[sl], deg2_hbm.at[pl.ds(c * N + s * 5000, 5000)])


RB = 400
GRID = N // RB
UB = NUM_USER // RB


def _mlp_body(pref, feat, w1, b1, w2, b2, deg_a, deg_b,
              x_out, y_out, di_out, di2_out):
    i = pl.program_id(0)

    @pl.when(i < UB)
    def _():
        x_out[...] = pref[...]

    @pl.when(i >= UB)
    def _():
        z = jnp.dot(feat[...], w1[...], preferred_element_type=_F32) + b1[...]
        z = jnp.where(z >= 0, z, 0.01 * z)
        x_out[...] = jnp.dot(z, w2[...], preferred_element_type=_F32) + b2[...]

    xb = x_out[...]
    nrm = jnp.sqrt(jnp.sum(xb * xb, axis=1, keepdims=True))
    xn = xb / jnp.maximum(nrm, 1e-12)
    x_out[...] = xn
    deg_sum = deg_a[0, 0, :] + deg_b[0, 0, :]
    dinv = jnp.where(deg_sum > 0, lax.rsqrt(deg_sum), 0.0)
    di_out[0, 0, :] = dinv
    di2_out[0, 0, :] = dinv * dinv
    y_out[...] = xn * dinv[:, None]


def _mlp(features, preference, W1, b1, W2, b2, deg2):
    deg3 = deg2.reshape(2, GRID, 1, RB)
    x, y, di, di2 = pl.pallas_call(
        _mlp_body,
        grid=(GRID,),
        in_specs=[
            pl.BlockSpec((RB, D), lambda i: (jnp.minimum(i, UB - 1), 0)),
            pl.BlockSpec((RB, DF), lambda i: (jnp.maximum(i - UB, 0), 0)),
            pl.BlockSpec((DF, DH), lambda i: (0, 0)),
            pl.BlockSpec((1, DH), lambda i: (0, 0)),
            pl.BlockSpec((DH, D), lambda i: (0, 0)),
            pl.BlockSpec((1, D), lambda i: (0, 0)),
            pl.BlockSpec((1, 1, RB), lambda i: (i, 0, 0)),
            pl.BlockSpec((1, 1, RB), lambda i: (i, 0, 0)),
        ],
        out_specs=[
            pl.BlockSpec((RB, D), lambda i: (i, 0)),
            pl.BlockSpec((RB, D), lambda i: (i, 0)),
            pl.BlockSpec((1, 1, RB), lambda i: (i, 0, 0)),
            pl.BlockSpec((1, 1, RB), lambda i: (i, 0, 0)),
        ],
        out_shape=[
            jax.ShapeDtypeStruct((N, D), _F32),
            jax.ShapeDtypeStruct((N, D), _F32),
            jax.ShapeDtypeStruct((GRID, 1, RB), _F32),
            jax.ShapeDtypeStruct((GRID, 1, RB), _F32),
        ],
    )(preference, features, W1, b1.reshape(1, DH), W2, b2.reshape(1, D),
      deg3[0], deg3[1])
    return x, y, di.reshape(N), di2.reshape(N)


EC3 = 50176
NCH3, NCH3_LAST = 392, 370
RT = 1568
RCH = RT // 16


def _zero_acc(s, zb, acc):
    def zbody(g, carry):
        zb[g // 4, pl.ds((g % 4) * 16, 16)] = _ZV16()
        return carry
    lax.fori_loop(0, 160 * 4, zbody, 0)
    for k in range(10):
        pltpu.sync_copy(zb, acc.at[pl.ds(s * 1600 + k * 160, 160)])


def _scatter_edges(src_hbm, gl_hbm, sl_hbm, cnt_hbm, cc, s,
                   gi_v, si_v, rows_v, cb_v, acc):
    for rg in range(2):
        w = 2 * s + rg
        pltpu.sync_copy(cnt_hbm.at[pl.ds(16 * w, 16)], cb_v)
        cb = cb_v[...]
        cnt = jnp.where(cc == 0, cb[0], cb[1])
        base = cc * LREG + w * EC1

        def body(j, carry):
            e0 = base + j * CH
            pltpu.sync_copy(gl_hbm.at[pl.ds(e0, CH)], gi_v)
            pltpu.sync_copy(sl_hbm.at[pl.ds(e0, CH)], si_v)
            pltpu.sync_copy(src_hbm.at[gi_v], rows_v)
            pltpu.sync_copy(rows_v, acc.at[si_v], add=True)
            return carry

        lax.fori_loop(0, cnt // CH, body, 0)


@functools.partial(
    pl.kernel,
    out_type=(jax.ShapeDtypeStruct((N, D), _F32),
              jax.ShapeDtypeStruct((N, D), _F32)),
    mesh=_mesh,
    compiler_params=pltpu.CompilerParams(use_tc_tiling_on_sc=False, needs_layout_passes=False),
    scratch_types=(pltpu.VMEM((CH,), jnp.int32),
                   pltpu.VMEM((CH,), jnp.int32),
                   pltpu.VMEM((CH, D), _F32),
                   pltpu.VMEM((16,), jnp.int32),
                   pltpu.VMEM((160, D), _F32),
                   pltpu.VMEM((RT,), _F32),
                   pltpu.VMEM((16, D), _F32),
                   pltpu.VMEM((16, D), _F32),
                   pltpu.VMEM_SHARED((ACC_ROWS, D), _F32)),
)
def _prop1(y_hbm, gl_hbm, sl_hbm, cnt_hbm, di2_hbm, s1_hbm, y2_hbm,
           gi_v, si_v, rows_v, cb_v, zb, dv, sv, yv, acc):
    cc = lax.axis_index("c")
    s = lax.axis_index("s")
    _zero_acc(s, zb, acc)
    plsc.subcore_barrier()
    _scatter_edges(y_hbm, gl_hbm, sl_hbm, cnt_hbm, cc, s, gi_v, si_v, rows_v, cb_v, acc)
    plsc.subcore_barrier()

    start = jnp.minimum(s * RT, HALF - RT)
    nbase = cc * HALF + start
    pltpu.sync_copy(di2_hbm.at[pl.ds(nbase, RT)], dv)

    def body(k, carry):
        r0 = start + k * 16
        g0 = nbase + k * 16
        pltpu.sync_copy(acc.at[pl.ds(r0, 16)], sv)
        d16 = dv[pl.ds(k * 16, 16)]
        for i in range(16):
            d = d16[i]
            for q in range(D // 16):
                sl = pl.ds(q * 16, 16)
                yv[i, sl] = sv[i, sl] * d
        pltpu.sync_copy(sv, s1_hbm.at[pl.ds(g0, 16)])
        pltpu.sync_copy(yv, y2_hbm.at[pl.ds(g0, 16)])
        return carry

    lax.fori_loop(0, RCH, body, 0)


@functools.partial(
    pl.kernel,
    out_type=jax.ShapeDtypeStruct((N, D), _F32),
    mesh=_mesh,
    compiler_params=pltpu.CompilerParams(use_tc_tiling_on_sc=False, needs_layout_passes=False),
    scratch_types=(pltpu.VMEM((CH,), jnp.int32),
                   pltpu.VMEM((CH,), jnp.int32),
                   pltpu.VMEM((CH, D), _F32),
                   pltpu.VMEM((16,), jnp.int32),
                   pltpu.VMEM((160, D), _F32),
                   pltpu.VMEM((RT,), _F32),
                   pltpu.VMEM((16, D), _F32),
                   pltpu.VMEM((16, D), _F32),
                   pltpu.VMEM((16, D), _F32),
                   pltpu.VMEM((16, D), _F32),
                   pltpu.VMEM_SHARED((ACC_ROWS, D), _F32)),
)
def _prop2(y2_hbm, gl_hbm, sl_hbm, cnt_hbm, di_hbm, x_hbm, s1_hbm, xhat_hbm,
           gi_v, si_v, rows_v, cb_v, zb, dv, sv, xv, s1v, ov, acc):
    cc = lax.axis_index("c")
    s = lax.axis_index("s")
    _zero_acc(s, zb, acc)
    plsc.subcore_barrier()
    _scatter_edges(y2_hbm, gl_hbm, sl_hbm, cnt_hbm, cc, s, gi_v, si_v, rows_v, cb_v, acc)
    plsc.subcore_barrier()

    start = jnp.minimum(s * RT, HALF - RT)
    nbase = cc * HALF + start
    pltpu.sync_copy(di_hbm.at[pl.ds(nbase, RT)], dv)

    def body(k, carry):
        r0 = start + k * 16
        g0 = nbase + k * 16
        pltpu.sync_copy(acc.at[pl.ds(r0, 16)], sv)
        pltpu.sync_copy(x_hbm.at[pl.ds(g0, 16)], xv)
        pltpu.sync_copy(s1_hbm.at[pl.ds(g0, 16)], s1v)
        d16 = dv[pl.ds(k * 16, 16)]
        for i in range(16):
            d = d16[i]
            for q in range(D // 16):
                sl = pl.ds(q * 16, 16)
                ov[i, sl] = xv[i, sl] + (sv[i, sl] + s1v[i, sl]) * d
        pltpu.sync_copy(ov, xhat_hbm.at[pl.ds(g0, 16)])
        return carry

    lax.fori_loop(0, RCH, body, 0)


def kernel(edge_index, features, preference, W1, b1, W2, b2):
    edge_flat = edge_index.reshape(2 * E)
    gl, slist, cnts, deg2 = _edge_prep(edge_flat)
    x, y, di, di2 = _mlp(features, preference, W1, b1, W2, b2, deg2)
    s1, y2 = _prop1(y, gl, slist, cnts, di2)
    x_hat = _prop2(y2, gl, slist, cnts, di, x, s1)
    return (x_hat, preference)


# speedup vs baseline: 13.1877x; 1.0411x over previous
import functools

import jax
import jax.numpy as jnp
from jax import lax
from jax.experimental import pallas as pl
from jax.experimental.pallas import tpu as pltpu
from jax.experimental.pallas import tpu_sc as plsc

NUM_USER = 10000
NUM_ITEM = 40000
N = NUM_USER + NUM_ITEM
DF = 128
DH = 256
D = 64
E = 800000

NC, NS = 2, 16
HALF = N // 2
DUMMY_MASK = 511
ACC_ROWS = 25600
CH = 128

_mesh = plsc.VectorSubcoreMesh(core_axis_name="c", subcore_axis_name="s",
                               num_cores=NC, num_subcores=NS)

_F32 = jnp.float32
_ZV16 = functools.partial(jnp.zeros, (16,), _F32)

EC1 = 25088
NCH1, NCH1_LAST = 196, 174


PAIR = 2 * CH
EC1R = EC1 + PAIR
LCAP = EC1R + 16
LREG = 32 * EC1R


@functools.partial(
    pl.kernel,
    out_type=(jax.ShapeDtypeStruct((2 * LREG,), jnp.int32),
              jax.ShapeDtypeStruct((2 * LREG,), jnp.int32),
              jax.ShapeDtypeStruct((512,), jnp.int32),
              jax.ShapeDtypeStruct((2 * N,), _F32)),
    mesh=_mesh,
    compiler_params=pltpu.CompilerParams(use_tc_tiling_on_sc=False, needs_layout_passes=False),
    scratch_types=(pltpu.VMEM((CH,), jnp.int32),
                   pltpu.VMEM((CH,), jnp.int32),
                   pltpu.VMEM((CH,), _F32),
                   pltpu.VMEM((LCAP,), jnp.int32),
                   pltpu.VMEM((LCAP,), jnp.int32),
                   pltpu.VMEM((LCAP,), jnp.int32),
                   pltpu.VMEM((LCAP,), jnp.int32),
                   pltpu.VMEM((16,), jnp.int32),
                   pltpu.VMEM((5008,), _F32),
                   pltpu.VMEM_SHARED((N,), _F32)),
)
def _edge_prep(edge_hbm, gl_hbm, sl_hbm, cnt_hbm, deg2_hbm,
               r_v, c_v, val_v, ga_v, sa_v, gb_v, sb_v, cw_v, z_v, deg_acc):
    c = lax.axis_index("c")
    s = lax.axis_index("s")
    wid = c * NS + s

    @pl.when(s < 10)
    def _():
        def zb(g, carry):
            z_v[pl.ds(g * 16, 16)] = _ZV16()
            return carry
        lax.fori_loop(0, 313, zb, 0)
        pltpu.sync_copy(z_v.at[pl.ds(0, 5000)], deg_acc.at[pl.ds(s * 5000, 5000)])

    plsc.subcore_barrier()

    nch = jnp.where(wid == NC * NS - 1, NCH1_LAST, NCH1)
    base_e = wid * EC1

    def body(j, offs):
        off_a, off_b = offs
        e0 = base_e + j * CH
        pltpu.sync_copy(edge_hbm.at[pl.ds(e0, CH)], r_v)
        pltpu.sync_copy(edge_hbm.at[pl.ds(E + e0, CH)], c_v)
        for g in range(CH // 16):
            sl = pl.ds(g * 16, 16)
            r = r_v[sl]
            cc = c_v[sl]
            keep = r != cc
            in_a = keep & (cc < HALF)
            in_b = keep & (cc >= HALF)
            cum_a = plsc.cumsum(jnp.where(in_a, 1, 0))
            cum_b = plsc.cumsum(jnp.where(in_b, 1, 0))
            dst_a = jnp.where(in_a, off_a + cum_a - 1, 0)
            dst_b = jnp.where(in_b, off_b + cum_b - 1, 0)
            plsc.store_scatter(ga_v, [dst_a], r, mask=in_a)
            plsc.store_scatter(sa_v, [dst_a], cc, mask=in_a)
            plsc.store_scatter(gb_v, [dst_b], r, mask=in_b)
            plsc.store_scatter(sb_v, [dst_b], cc - HALF, mask=in_b)
            off_a = off_a + cum_a[15]
            off_b = off_b + cum_b[15]
            val_v[sl] = jnp.where(keep, 1.0, 0.0).astype(_F32)
        pltpu.sync_copy(val_v, deg_acc.at[r_v], add=True)
        return (off_a, off_b)

    off_a, off_b = lax.fori_loop(0, nch, body, (jnp.int32(0), jnp.int32(0)))

    iota = lax.broadcasted_iota(jnp.int32, (16,), 0)

    def _pad(off, g_ref, s_ref):
        pad_to = ((off + PAIR - 1) // PAIR) * PAIR

        def pbody(k, o):
            g_ref[pl.ds(o, 16)] = jnp.zeros((16,), jnp.int32)
            s_ref[pl.ds(o, 16)] = HALF + ((iota * 31 + o) & DUMMY_MASK)
            return o + 16

        lax.fori_loop(0, (pad_to - off + 15) // 16, pbody, off)
        return pad_to

    cnt_a = _pad(off_a, ga_v, sa_v)
    cnt_b = _pad(off_b, gb_v, sb_v)

    base_a = wid * EC1R
    base_b = LREG + wid * EC1R
    pltpu.sync_copy(ga_v.at[pl.ds(0, EC1R)], gl_hbm.at[pl.ds(base_a, EC1R)])
    pltpu.sync_copy(sa_v.at[pl.ds(0, EC1R)], sl_hbm.at[pl.ds(base_a, EC1R)])
    pltpu.sync_copy(gb_v.at[pl.ds(0, EC1R)], gl_hbm.at[pl.ds(base_b, EC1R)])
    pltpu.sync_copy(sb_v.at[pl.ds(0, EC1R)], sl_hbm.at[pl.ds(base_b, EC1R)])
    cw_v[pl.ds(0, 16)] = jnp.where(iota == 0, cnt_a, jnp.where(iota == 1, cnt_b, 0))
    pltpu.sync_copy(cw_v, cnt_hbm.at[pl.ds(16 * wid, 16)])

    plsc.subcore_barrier()

    @pl.when(s < 10)
    def _():
        sl = pl.ds(0, 5000)
        pltpu.sync_copy(deg_acc.at[pl.ds(s * 5000, 5000)], z_v.at[sl])
        pltpu.sync_copy(z_v.at[sl], deg2_hbm.at[pl.ds(c * N + s * 5000, 5000)])


RB = 400
GRID = N // RB
UB = NUM_USER // RB


def _mlp_body(pref, feat, w1, b1, w2, b2, deg_a, deg_b,
              x_out, y_out, di_out, di2_out):
    i = pl.program_id(0)

    @pl.when(i < UB)
    def _():
        x_out[...] = pref[...]

    @pl.when(i >= UB)
    def _():
        z = jnp.dot(feat[...], w1[...], preferred_element_type=_F32) + b1[...]
        z = jnp.where(z >= 0, z, 0.01 * z)
        x_out[...] = jnp.dot(z, w2[...], preferred_element_type=_F32) + b2[...]

    xb = x_out[...]
    nrm = jnp.sqrt(jnp.sum(xb * xb, axis=1, keepdims=True))
    xn = xb / jnp.maximum(nrm, 1e-12)
    x_out[...] = xn
    deg_sum = deg_a[0, 0, :] + deg_b[0, 0, :]
    dinv = jnp.where(deg_sum > 0, lax.rsqrt(deg_sum), 0.0)
    di_out[0, 0, :] = dinv
    di2_out[0, 0, :] = dinv * dinv
    y_out[...] = xn * dinv[:, None]


def _mlp(features, preference, W1, b1, W2, b2, deg2):
    deg3 = deg2.reshape(2, GRID, 1, RB)
    x, y, di, di2 = pl.pallas_call(
        _mlp_body,
        grid=(GRID,),
        in_specs=[
            pl.BlockSpec((RB, D), lambda i: (jnp.minimum(i, UB - 1), 0)),
            pl.BlockSpec((RB, DF), lambda i: (jnp.maximum(i - UB, 0), 0)),
            pl.BlockSpec((DF, DH), lambda i: (0, 0)),
            pl.BlockSpec((1, DH), lambda i: (0, 0)),
            pl.BlockSpec((DH, D), lambda i: (0, 0)),
            pl.BlockSpec((1, D), lambda i: (0, 0)),
            pl.BlockSpec((1, 1, RB), lambda i: (i, 0, 0)),
            pl.BlockSpec((1, 1, RB), lambda i: (i, 0, 0)),
        ],
        out_specs=[
            pl.BlockSpec((RB, D), lambda i: (i, 0)),
            pl.BlockSpec((RB, D), lambda i: (i, 0)),
            pl.BlockSpec((1, 1, RB), lambda i: (i, 0, 0)),
            pl.BlockSpec((1, 1, RB), lambda i: (i, 0, 0)),
        ],
        out_shape=[
            jax.ShapeDtypeStruct((N, D), _F32),
            jax.ShapeDtypeStruct((N, D), _F32),
            jax.ShapeDtypeStruct((GRID, 1, RB), _F32),
            jax.ShapeDtypeStruct((GRID, 1, RB), _F32),
        ],
    )(preference, features, W1, b1.reshape(1, DH), W2, b2.reshape(1, D),
      deg3[0], deg3[1])
    return x, y, di.reshape(N), di2.reshape(N)


EC3 = 50176
NCH3, NCH3_LAST = 392, 370
RT = 1568
RCH = RT // 16


def _zero_acc(s, zb, acc):
    def zbody(g, carry):
        zb[g // 4, pl.ds((g % 4) * 16, 16)] = _ZV16()
        return carry
    lax.fori_loop(0, 64 * 4, zbody, 0)
    for k in range(25):
        pltpu.sync_copy(zb, acc.at[pl.ds(s * 1600 + k * 64, 64)])


def _scatter_edges(src_hbm, gl_hbm, sl_hbm, cnt_hbm, cc, s,
                   gi2, si2, rows2, cb_v, sem_g, sem_s, acc):
    for rg in range(2):
        w = 2 * s + rg
        pltpu.sync_copy(cnt_hbm.at[pl.ds(16 * w, 16)], cb_v)
        cb = cb_v[...]
        cnt = jnp.where(cc == 0, cb[0], cb[1])
        base = cc * LREG + w * EC1R
        npair = cnt // PAIR
        nch = npair * 2

        @pl.when(npair > 0)
        def _():
            pltpu.sync_copy(gl_hbm.at[pl.ds(base, PAIR)], gi2.at[0])
            pltpu.sync_copy(sl_hbm.at[pl.ds(base, CH)], si2.at[0, 0])
            pltpu.sync_copy(sl_hbm.at[pl.ds(base + CH, CH)], si2.at[0, 1])
            pltpu.async_copy(src_hbm.at[gi2.at[0, pl.ds(0, CH)]],
                             rows2.at[0], sem_g)

        def pair_body(t, carry):
            tp = t % 2
            for b in range(2):
                j = 2 * t + b
                pltpu.make_async_copy(src_hbm.at[pl.ds(0, CH)],
                                      rows2.at[b], sem_g).wait()
                pltpu.async_copy(rows2.at[b], acc.at[si2.at[tp, b]],
                                 sem_s, add=True)
                if b == 1:
                    @pl.when(t + 1 < npair)
                    def _():
                        e1 = base + (t + 1) * PAIR
                        pltpu.sync_copy(gl_hbm.at[pl.ds(e1, PAIR)],
                                        gi2.at[1 - tp])
                        pltpu.sync_copy(sl_hbm.at[pl.ds(e1, CH)],
                                        si2.at[1 - tp, 0])
                        pltpu.sync_copy(sl_hbm.at[pl.ds(e1 + CH, CH)],
                                        si2.at[1 - tp, 1])

                @pl.when(j >= 1)
                def _():
                    pltpu.make_async_copy(src_hbm.at[pl.ds(0, CH)],
                                          rows2.at[1 - b], sem_s).wait()

                @pl.when(j + 1 < nch)
                def _():
                    if b == 0:
                        idxref = gi2.at[tp, pl.ds(CH, CH)]
                    else:
                        idxref = gi2.at[1 - tp, pl.ds(0, CH)]
                    pltpu.async_copy(src_hbm.at[idxref], rows2.at[1 - b], sem_g)
            return carry

        lax.fori_loop(0, npair, pair_body, 0)

        @pl.when(npair > 0)
        def _():
            pltpu.make_async_copy(src_hbm.at[pl.ds(0, CH)],
                                  rows2.at[1], sem_s).wait()


@functools.partial(
    pl.kernel,
    out_type=(jax.ShapeDtypeStruct((N, D), _F32),
              jax.ShapeDtypeStruct((N, D), _F32)),
    mesh=_mesh,
    compiler_params=pltpu.CompilerParams(use_tc_tiling_on_sc=False, needs_layout_passes=False),
    scratch_types=(pltpu.VMEM((2, PAIR), jnp.int32),
                   pltpu.VMEM((2, 2, CH), jnp.int32),
                   pltpu.VMEM((2, CH, D), _F32),
                   pltpu.VMEM((16,), jnp.int32),
                   pltpu.SemaphoreType.DMA,
                   pltpu.SemaphoreType.DMA,
                   pltpu.VMEM((64, D), _F32),
                   pltpu.VMEM((RT,), _F32),
                   pltpu.VMEM((16, D), _F32),
                   pltpu.VMEM((16, D), _F32),
                   pltpu.VMEM_SHARED((ACC_ROWS, D), _F32)),
)
def _prop1(y_hbm, gl_hbm, sl_hbm, cnt_hbm, di2_hbm, s1_hbm, y2_hbm,
           gi2, si2, rows2, cb_v, sem_g, sem_s, zb, dv, sv, yv, acc):
    cc = lax.axis_index("c")
    s = lax.axis_index("s")
    _zero_acc(s, zb, acc)
    plsc.subcore_barrier()
    _scatter_edges(y_hbm, gl_hbm, sl_hbm, cnt_hbm, cc, s,
                   gi2, si2, rows2, cb_v, sem_g, sem_s, acc)
    plsc.subcore_barrier()

    start = jnp.minimum(s * RT, HALF - RT)
    nbase = cc * HALF + start
    pltpu.sync_copy(di2_hbm.at[pl.ds(nbase, RT)], dv)

    def body(k, carry):
        r0 = start + k * 16
        g0 = nbase + k * 16
        pltpu.sync_copy(acc.at[pl.ds(r0, 16)], sv)
        d16 = dv[pl.ds(k * 16, 16)]
        for i in range(16):
            d = d16[i]
            for q in range(D // 16):
                sl = pl.ds(q * 16, 16)
                yv[i, sl] = sv[i, sl] * d
        pltpu.sync_copy(sv, s1_hbm.at[pl.ds(g0, 16)])
        pltpu.sync_copy(yv, y2_hbm.at[pl.ds(g0, 16)])
        return carry

    lax.fori_loop(0, RCH, body, 0)


@functools.partial(
    pl.kernel,
    out_type=jax.ShapeDtypeStruct((N, D), _F32),
    mesh=_mesh,
    compiler_params=pltpu.CompilerParams(use_tc_tiling_on_sc=False, needs_layout_passes=False),
    scratch_types=(pltpu.VMEM((2, PAIR), jnp.int32),
                   pltpu.VMEM((2, 2, CH), jnp.int32),
                   pltpu.VMEM((2, CH, D), _F32),
                   pltpu.VMEM((16,), jnp.int32),
                   pltpu.SemaphoreType.DMA,
                   pltpu.SemaphoreType.DMA,
                   pltpu.VMEM((64, D), _F32),
                   pltpu.VMEM((RT,), _F32),
                   pltpu.VMEM((16, D), _F32),
                   pltpu.VMEM((16, D), _F32),
                   pltpu.VMEM((16, D), _F32),
                   pltpu.VMEM((16, D), _F32),
                   pltpu.VMEM_SHARED((ACC_ROWS, D), _F32)),
)
def _prop2(y2_hbm, gl_hbm, sl_hbm, cnt_hbm, di_hbm, x_hbm, s1_hbm, xhat_hbm,
           gi2, si2, rows2, cb_v, sem_g, sem_s, zb, dv, sv, xv, s1v, ov, acc):
    cc = lax.axis_index("c")
    s = lax.axis_index("s")
    _zero_acc(s, zb, acc)
    plsc.subcore_barrier()
    _scatter_edges(y2_hbm, gl_hbm, sl_hbm, cnt_hbm, cc, s,
                   gi2, si2, rows2, cb_v, sem_g, sem_s, acc)
    plsc.subcore_barrier()

    start = jnp.minimum(s * RT, HALF - RT)
    nbase = cc * HALF + start
    pltpu.sync_copy(di_hbm.at[pl.ds(nbase, RT)], dv)

    def body(k, carry):
        r0 = start + k * 16
        g0 = nbase + k * 16
        pltpu.sync_copy(acc.at[pl.ds(r0, 16)], sv)
        pltpu.sync_copy(x_hbm.at[pl.ds(g0, 16)], xv)
        pltpu.sync_copy(s1_hbm.at[pl.ds(g0, 16)], s1v)
        d16 = dv[pl.ds(k * 16, 16)]
        for i in range(16):
            d = d16[i]
            for q in range(D // 16):
                sl = pl.ds(q * 16, 16)
                ov[i, sl] = xv[i, sl] + (sv[i, sl] + s1v[i, sl]) * d
        pltpu.sync_copy(ov, xhat_hbm.at[pl.ds(g0, 16)])
        return carry

    lax.fori_loop(0, RCH, body, 0)


def kernel(edge_index, features, preference, W1, b1, W2, b2):
    edge_flat = edge_index.reshape(2 * E)
    gl, slist, cnts, deg2 = _edge_prep(edge_flat)
    x, y, di, di2 = _mlp(features, preference, W1, b1, W2, b2, deg2)
    s1, y2 = _prop1(y, gl, slist, cnts, di2)
    x_hat = _prop2(y2, gl, slist, cnts, di, x, s1)
    return (x_hat, preference)
